# baseline (speedup 1.0000x reference)
import functools

import jax
import jax.numpy as jnp
from jax import lax
from jax.experimental import pallas as pl
from jax.experimental.pallas import tpu as pltpu
from jax.experimental.pallas import tpu_sc as plsc

K = 16
BM = 256
_BIG = 3.0e38

_NC, _NS, _L = 2, 16, 16
_NW = _NC * _NS
_CH = 8


_SB = 16
_LDEPTH = 5
_BIGI = 1 << 30


def _extract16(d2, n_sent):
    rows = d2.shape[0]
    col_ids = lax.broadcasted_iota(jnp.int32, d2.shape, 1)
    cols = []
    for _ in range(K):
        m = jnp.min(d2, axis=1, keepdims=True)
        cand = jnp.where(d2 == m, col_ids, n_sent)
        j = jnp.min(cand, axis=1, keepdims=True)
        cols.append(j)
        d2 = jnp.where(cand == j, _BIG, d2)
    return jnp.concatenate(cols, axis=1)


def _ag_kernel(x_ref, w_ref, b_ref, a_ref, g_ref):
    xb = x_ref[...]
    w1 = w_ref[0:128, :]
    w2 = w_ref[128:256, :]
    g_ref[...] = jnp.dot(xb, w2, preferred_element_type=jnp.float32,
                         precision=lax.Precision.HIGHEST)
    a_ref[...] = (jnp.dot(xb, w1 - w2, preferred_element_type=jnp.float32,
                          precision=lax.Precision.HIGHEST)
                  + b_ref[...])


def _ag_stage(x, w, b2d):
    n, d = x.shape
    bm = 512
    return pl.pallas_call(
        _ag_kernel,
        grid=(pl.cdiv(n, bm),),
        in_specs=[
            pl.BlockSpec((bm, d), lambda i: (i, 0)),
            pl.BlockSpec((2 * d, d), lambda i: (0, 0)),
            pl.BlockSpec((1, d), lambda i: (0, 0)),
        ],
        out_specs=[
            pl.BlockSpec((bm, d), lambda i: (i, 0)),
            pl.BlockSpec((bm, d), lambda i: (i, 0)),
        ],
        out_shape=[
            jax.ShapeDtypeStruct((n, d), jnp.float32),
            jax.ShapeDtypeStruct((n, d), jnp.float32),
        ],
    )(x, w, b2d)


def _knn_tc_kernel(row0, n_hi, x_ref, xt_ref, idx_ref):
    i = pl.program_id(0)
    npad = xt_ref.shape[1]
    nchunks = npad // 128
    xb = x_ref[...]
    xt = xt_ref[...]

    sq_j = jnp.sum(xt * xt, axis=0, keepdims=True)
    dot = lax.dot_general(
        xb, xt, (((1,), (0,)), ((), ())),
        preferred_element_type=jnp.float32,
        precision=lax.Precision.DEFAULT)

    row_ids = row0 + i * BM + lax.broadcasted_iota(jnp.int32, (BM, 1), 0)

    lane = lax.broadcasted_iota(jnp.int32, (_SB, 128), 1)
    idx_parts, bad_parts = [], []
    for s in range(BM // _SB):
        r0 = s * _SB
        rids_s = row_ids[r0:r0 + _SB, :]
        vals = [jnp.full((_SB, 128), _BIG, jnp.float32) for _ in range(_LDEPTH)]
        chks = [jnp.zeros((_SB, 128), jnp.int32) for _ in range(_LDEPTH)]
        for c in range(nchunks):
            cs = c * 128
            v = sq_j[:, cs:cs + 128] - 2.0 * dot[r0:r0 + _SB, cs:cs + 128]
            lt = [v < vals[t] for t in range(_LDEPTH)]
            cc = jnp.full((_SB, 128), c, jnp.int32)
            new_vals, new_chks = [], []
            for t in range(_LDEPTH - 1, 0, -1):
                new_vals.append(jnp.where(lt[t],
                                          jnp.where(lt[t - 1], vals[t - 1], v),
                                          vals[t]))
                new_chks.append(jnp.where(lt[t],
                                          jnp.where(lt[t - 1], chks[t - 1], cc),
                                          chks[t]))
            new_vals.append(jnp.where(lt[0], v, vals[0]))
            new_chks.append(jnp.where(lt[0], cc, chks[0]))
            vals = new_vals[::-1]
            chks = new_chks[::-1]
        cv = jnp.concatenate(vals, axis=1)
        cid = jnp.concatenate([chks[t] * 128 + lane for t in range(_LDEPTH)],
                              axis=1)
        cols = []
        m = None
        for _ in range(K + 1):
            m = jnp.min(cv, axis=1, keepdims=True)
            sel = cv == m
            candid = jnp.where(sel, cid, _BIGI)
            j = jnp.min(candid, axis=1, keepdims=True)
            cols.append(j)
            cv = jnp.where(sel & (cid == j), _BIG, cv)
        found = cols[0] == rids_s
        outs = []
        for t in range(K):
            outs.append(jnp.where(found, cols[t + 1], cols[t]))
            if t + 1 < K:
                found = found | (cols[t + 1] == rids_s)
        idx_parts.append(jnp.concatenate(outs, axis=1))
        bad = jnp.max(jnp.where(vals[_LDEPTH - 1] <= m, 1, 0),
                      axis=1, keepdims=True)
        bad_parts.append(bad)
    idx_cand = jnp.concatenate(idx_parts, axis=0)
    badv = jnp.concatenate(bad_parts, axis=0)

    def _fallback():
        col_ids = lax.broadcasted_iota(jnp.int32, (BM, npad), 1)
        d2 = sq_j - 2.0 * dot
        d2 = jnp.where(col_ids == row_ids, _BIG, d2)
        return _extract16(d2, npad)

    ok = jnp.all((badv == 0) | (row_ids >= n_hi))
    idx_ref[...] = lax.cond(ok, lambda: idx_cand, _fallback)


def _knn_stage(x_part, xt_pad, row0):
    nrows, d = x_part.shape
    npad = xt_pad.shape[1]
    grid = (pl.cdiv(nrows, BM),)
    return pl.pallas_call(
        functools.partial(_knn_tc_kernel, row0, row0 + nrows),
        grid=grid,
        in_specs=[
            pl.BlockSpec((BM, d), lambda i: (i, 0)),
            pl.BlockSpec((d, npad), lambda i: (0, 0)),
        ],
        out_specs=pl.BlockSpec((BM, K), lambda i: (i, 0)),
        out_shape=jax.ShapeDtypeStruct((nrows, K), jnp.int32),
    )(x_part, xt_pad)


def _gather_max_stage(idx2d, a_pad, g, n_pad, d):
    npw = n_pad // _NW
    nchunks = npw // _CH
    cstride = ((nchunks + 7) // 8) * 8
    mesh = plsc.VectorSubcoreMesh(core_axis_name="c", subcore_axis_name="s")

    @functools.partial(
        pl.kernel,
        mesh=mesh,
        out_type=jax.ShapeDtypeStruct((n_pad, d), jnp.float32),
        scratch_types=[
            pltpu.VMEM((cstride, _CH * K), jnp.int32),
            pltpu.VMEM((_CH * K, d), jnp.float32),
            pltpu.VMEM((_CH * K, d), jnp.float32),
            pltpu.VMEM((npw, d), jnp.float32),
            pltpu.VMEM((npw, d), jnp.float32),
            pltpu.SemaphoreType.DMA,
            pltpu.SemaphoreType.DMA,
        ],
    )
    def sck(idx_hbm, a_hbm, g_hbm, out_hbm, idx_v, g_v0, g_v1, a_v, o_v,
            sem0, sem1):
        wid = lax.axis_index("s") * _NC + lax.axis_index("c")
        base = wid * npw

        pltpu.sync_copy(idx_hbm.at[pl.ds(wid * cstride, cstride)], idx_v)
        pltpu.sync_copy(a_hbm.at[pl.ds(base, npw)], a_v)

        gbufs = (g_v0, g_v1)
        sems = (sem0, sem1)

        def start(ci, b):
            pltpu.make_async_copy(g_hbm.at[idx_v.at[ci]], gbufs[b],
                                  sems[b]).start()

        def compute(ci, b):
            g_v = gbufs[b]
            pltpu.make_async_copy(g_hbm.at[idx_v.at[ci]], g_v, sems[b]).wait()

            @pl.loop(0, _CH)
            def _(nn):
                r0 = nn * K
                orow = ci * _CH + nn
                for v in range(d // _L):
                    sl = pl.ds(v * _L, _L)
                    acc = jnp.maximum(g_v[r0, sl], g_v[r0 + 1, sl])
                    for r in range(2, K):
                        acc = jnp.maximum(acc, g_v[r0 + r, sl])
                    o_v[orow, sl] = jnp.maximum(a_v[orow, sl] + acc, 0.0)

        start(0, 0)
        start(1, 1)

        @pl.loop(0, nchunks // 2)
        def _(gg):
            c0 = gg * 2
            for b in range(2):
                ci = c0 + b
                compute(ci, b)

                @pl.when(ci + 2 < nchunks)
                def _():
                    start(ci + 2, b)

        pltpu.sync_copy(o_v, out_hbm.at[pl.ds(base, npw)])

    return sck(idx2d, a_pad, g)


def kernel(x, edge_index, edge_attr, W, b):
    del edge_index, edge_attr
    n, d = x.shape
    npad = ((n + 127) // 128) * 128
    xt_pad = jnp.pad(x.T, ((0, 0), (0, npad - n)), constant_values=1000.0)
    b2d = b.reshape(1, d)

    a, g = _ag_stage(x, W, b2d)

    n_pad = ((n + _NW * _CH - 1) // (_NW * _CH)) * (_NW * _CH)
    a_pad = jnp.pad(a, ((0, n_pad - n), (0, 0)))

    half = n_pad // 2
    outs = []
    for s in range(2):
        r0 = s * half
        r1 = min((s + 1) * half, n)
        idx_s = _knn_stage(x[r0:r1], xt_pad, r0)
        nchunks = half // _NW // _CH
        cstride = ((nchunks + 7) // 8) * 8
        idx2d_s = jnp.pad(idx_s.reshape(-1),
                          (0, (half - (r1 - r0)) * K)).reshape(
                              _NW, nchunks, _CH * K)
        idx2d_s = jnp.pad(idx2d_s, ((0, 0), (0, cstride - nchunks),
                                    (0, 0))).reshape(-1, _CH * K)
        outs.append(_gather_max_stage(idx2d_s, a_pad[r0:r0 + half], g,
                                      half, d))
    return jnp.concatenate(outs, axis=0)[:n]


# speedup vs baseline: 1.0244x; 1.0244x over previous
import functools

import jax
import jax.numpy as jnp
from jax import lax
from jax.experimental import pallas as pl
from jax.experimental.pallas import tpu as pltpu
from jax.experimental.pallas import tpu_sc as plsc

K = 16
BM = 256
_BIG = 3.0e38

_NC, _NS, _L = 2, 16, 16
_NW = _NC * _NS
_CH = 8


_SB = 16
_LDEPTH = 5
_BIGI = 1 << 30


def _extract16(d2, n_sent):
    rows = d2.shape[0]
    col_ids = lax.broadcasted_iota(jnp.int32, d2.shape, 1)
    cols = []
    for _ in range(K):
        m = jnp.min(d2, axis=1, keepdims=True)
        cand = jnp.where(d2 == m, col_ids, n_sent)
        j = jnp.min(cand, axis=1, keepdims=True)
        cols.append(j)
        d2 = jnp.where(cand == j, _BIG, d2)
    return jnp.concatenate(cols, axis=1)


def _ag_kernel(x_ref, w_ref, b_ref, a_ref, g_ref):
    xb = x_ref[...]
    w1 = w_ref[0:128, :]
    w2 = w_ref[128:256, :]
    g_ref[...] = jnp.dot(xb, w2, preferred_element_type=jnp.float32,
                         precision=lax.Precision.HIGHEST)
    a_ref[...] = (jnp.dot(xb, w1 - w2, preferred_element_type=jnp.float32,
                          precision=lax.Precision.HIGHEST)
                  + b_ref[...])


def _ag_stage(x, w, b2d):
    n, d = x.shape
    bm = 512
    return pl.pallas_call(
        _ag_kernel,
        grid=(pl.cdiv(n, bm),),
        in_specs=[
            pl.BlockSpec((bm, d), lambda i: (i, 0)),
            pl.BlockSpec((2 * d, d), lambda i: (0, 0)),
            pl.BlockSpec((1, d), lambda i: (0, 0)),
        ],
        out_specs=[
            pl.BlockSpec((bm, d), lambda i: (i, 0)),
            pl.BlockSpec((bm, d), lambda i: (i, 0)),
        ],
        out_shape=[
            jax.ShapeDtypeStruct((n, d), jnp.float32),
            jax.ShapeDtypeStruct((n, d), jnp.float32),
        ],
    )(x, w, b2d)


def _knn_tc_kernel(row0, n_hi, x_ref, xt_ref, idx_ref):
    i = pl.program_id(0)
    npad = xt_ref.shape[1]
    nchunks = npad // 128
    xb = x_ref[...]
    xt = xt_ref[...]

    sq_j = jnp.sum(xt * xt, axis=0, keepdims=True)
    dot = lax.dot_general(
        xb, xt, (((1,), (0,)), ((), ())),
        preferred_element_type=jnp.float32,
        precision=lax.Precision.DEFAULT)

    row_ids = row0 + i * BM + lax.broadcasted_iota(jnp.int32, (BM, 1), 0)

    lane = lax.broadcasted_iota(jnp.int32, (_SB, 128), 1)
    idx_parts, bad_parts = [], []
    for s in range(BM // _SB):
        r0 = s * _SB
        rids_s = row_ids[r0:r0 + _SB, :]
        vals = [jnp.full((_SB, 128), _BIG, jnp.float32) for _ in range(_LDEPTH)]
        chks = [jnp.zeros((_SB, 128), jnp.int32) for _ in range(_LDEPTH)]
        for c in range(nchunks):
            cs = c * 128
            v = sq_j[:, cs:cs + 128] - 2.0 * dot[r0:r0 + _SB, cs:cs + 128]
            lt = [v < vals[t] for t in range(_LDEPTH)]
            cc = jnp.full((_SB, 128), c, jnp.int32)
            new_vals, new_chks = [], []
            for t in range(_LDEPTH - 1, 0, -1):
                new_vals.append(jnp.where(lt[t],
                                          jnp.where(lt[t - 1], vals[t - 1], v),
                                          vals[t]))
                new_chks.append(jnp.where(lt[t],
                                          jnp.where(lt[t - 1], chks[t - 1], cc),
                                          chks[t]))
            new_vals.append(jnp.where(lt[0], v, vals[0]))
            new_chks.append(jnp.where(lt[0], cc, chks[0]))
            vals = new_vals[::-1]
            chks = new_chks[::-1]
        cv = jnp.concatenate(vals, axis=1)
        cid = jnp.concatenate([chks[t] * 128 + lane for t in range(_LDEPTH)],
                              axis=1)
        cols = []
        m = None
        for _ in range(K + 1):
            m = jnp.min(cv, axis=1, keepdims=True)
            sel = cv == m
            candid = jnp.where(sel, cid, _BIGI)
            j = jnp.min(candid, axis=1, keepdims=True)
            cols.append(j)
            cv = jnp.where(sel & (cid == j), _BIG, cv)
        found = cols[0] == rids_s
        outs = []
        for t in range(K):
            outs.append(jnp.where(found, cols[t + 1], cols[t]))
            if t + 1 < K:
                found = found | (cols[t + 1] == rids_s)
        idx_parts.append(jnp.concatenate(outs, axis=1))
        bad = jnp.max(jnp.where(vals[_LDEPTH - 1] <= m, 1, 0),
                      axis=1, keepdims=True)
        bad_parts.append(bad)
    idx_cand = jnp.concatenate(idx_parts, axis=0)
    badv = jnp.concatenate(bad_parts, axis=0)

    def _fallback():
        col_ids = lax.broadcasted_iota(jnp.int32, (BM, npad), 1)
        d2 = sq_j - 2.0 * dot
        d2 = jnp.where(col_ids == row_ids, _BIG, d2)
        return _extract16(d2, npad)

    ok = jnp.all((badv == 0) | (row_ids >= n_hi))
    idx_ref[...] = lax.cond(ok, lambda: idx_cand, _fallback)


def _knn_stage(x_part, xt_pad, row0):
    nrows, d = x_part.shape
    npad = xt_pad.shape[1]
    grid = (pl.cdiv(nrows, BM),)
    return pl.pallas_call(
        functools.partial(_knn_tc_kernel, row0, row0 + nrows),
        grid=grid,
        in_specs=[
            pl.BlockSpec((BM, d), lambda i: (i, 0)),
            pl.BlockSpec((d, npad), lambda i: (0, 0)),
        ],
        out_specs=pl.BlockSpec((BM, K), lambda i: (i, 0)),
        out_shape=jax.ShapeDtypeStruct((nrows, K), jnp.int32),
    )(x_part, xt_pad)


def _gather_max_stage(idx2d, a_pad, g, n_pad, d):
    npw = n_pad // _NW
    nchunks = npw // _CH
    cstride = ((nchunks + 7) // 8) * 8
    mesh = plsc.VectorSubcoreMesh(core_axis_name="c", subcore_axis_name="s")

    @functools.partial(
        pl.kernel,
        mesh=mesh,
        out_type=jax.ShapeDtypeStruct((n_pad, d), jnp.float32),
        scratch_types=[
            pltpu.VMEM((cstride, _CH * K), jnp.int32),
            pltpu.VMEM((_CH * K, d), jnp.float32),
            pltpu.VMEM((_CH * K, d), jnp.float32),
            pltpu.VMEM((npw, d), jnp.float32),
            pltpu.VMEM((npw, d), jnp.float32),
            pltpu.SemaphoreType.DMA,
            pltpu.SemaphoreType.DMA,
        ],
    )
    def sck(idx_hbm, a_hbm, g_hbm, out_hbm, idx_v, g_v0, g_v1, a_v, o_v,
            sem0, sem1):
        wid = lax.axis_index("s") * _NC + lax.axis_index("c")
        base = wid * npw

        pltpu.sync_copy(idx_hbm.at[pl.ds(wid * cstride, cstride)], idx_v)
        pltpu.sync_copy(a_hbm.at[pl.ds(base, npw)], a_v)

        gbufs = (g_v0, g_v1)
        sems = (sem0, sem1)

        def start(ci, b):
            pltpu.make_async_copy(g_hbm.at[idx_v.at[ci]], gbufs[b],
                                  sems[b]).start()

        def compute(ci, b):
            g_v = gbufs[b]
            pltpu.make_async_copy(g_hbm.at[idx_v.at[ci]], g_v, sems[b]).wait()

            @pl.loop(0, _CH)
            def _(nn):
                r0 = nn * K
                orow = ci * _CH + nn
                for v in range(d // _L):
                    sl = pl.ds(v * _L, _L)
                    acc = jnp.maximum(g_v[r0, sl], g_v[r0 + 1, sl])
                    for r in range(2, K):
                        acc = jnp.maximum(acc, g_v[r0 + r, sl])
                    o_v[orow, sl] = jnp.maximum(a_v[orow, sl] + acc, 0.0)

        start(0, 0)
        start(1, 1)

        @pl.loop(0, nchunks // 2)
        def _(gg):
            c0 = gg * 2
            for b in range(2):
                ci = c0 + b
                compute(ci, b)

                @pl.when(ci + 2 < nchunks)
                def _():
                    start(ci + 2, b)

        pltpu.sync_copy(o_v, out_hbm.at[pl.ds(base, npw)])

    return sck(idx2d, a_pad, g)


def kernel(x, edge_index, edge_attr, W, b):
    del edge_index, edge_attr
    n, d = x.shape
    npad = ((n + 127) // 128) * 128
    xt_pad = jnp.pad(x.T, ((0, 0), (0, npad - n)), constant_values=1000.0)
    b2d = b.reshape(1, d)

    a, g = _ag_stage(x, W, b2d)

    n_pad = ((n + _NW * _CH - 1) // (_NW * _CH)) * (_NW * _CH)
    a_pad = jnp.pad(a, ((0, n_pad - n), (0, 0)))

    nsplit = 1
    half = n_pad // nsplit
    outs = []
    for s in range(nsplit):
        r0 = s * half
        r1 = min((s + 1) * half, n)
        idx_s = _knn_stage(x[r0:r1], xt_pad, r0)
        nchunks = half // _NW // _CH
        cstride = ((nchunks + 7) // 8) * 8
        idx2d_s = jnp.pad(idx_s.reshape(-1),
                          (0, (half - (r1 - r0)) * K)).reshape(
                              _NW, nchunks, _CH * K)
        idx2d_s = jnp.pad(idx2d_s, ((0, 0), (0, cstride - nchunks),
                                    (0, 0))).reshape(-1, _CH * K)
        outs.append(_gather_max_stage(idx2d_s, a_pad[r0:r0 + half], g,
                                      half, d))
    return jnp.concatenate(outs, axis=0)[:n]


# speedup vs baseline: 1.0256x; 1.0011x over previous
import functools

import jax
import jax.numpy as jnp
from jax import lax
from jax.experimental import pallas as pl
from jax.experimental.pallas import tpu as pltpu
from jax.experimental.pallas import tpu_sc as plsc

K = 16
BM = 256
_BIG = 3.0e38

_NC, _NS, _L = 2, 16, 16
_NW = _NC * _NS
_CH = 8


_SB = 16
_LDEPTH = 5
_BIGI = 1 << 30


def _extract16(d2, n_sent):
    rows = d2.shape[0]
    col_ids = lax.broadcasted_iota(jnp.int32, d2.shape, 1)
    cols = []
    for _ in range(K):
        m = jnp.min(d2, axis=1, keepdims=True)
        cand = jnp.where(d2 == m, col_ids, n_sent)
        j = jnp.min(cand, axis=1, keepdims=True)
        cols.append(j)
        d2 = jnp.where(cand == j, _BIG, d2)
    return jnp.concatenate(cols, axis=1)


def _ag_kernel(x_ref, w_ref, b_ref, a_ref, g_ref):
    xb = x_ref[...]
    w1 = w_ref[0:128, :]
    w2 = w_ref[128:256, :]
    g_ref[...] = jnp.dot(xb, w2, preferred_element_type=jnp.float32,
                         precision=lax.Precision.HIGHEST)
    a_ref[...] = (jnp.dot(xb, w1 - w2, preferred_element_type=jnp.float32,
                          precision=lax.Precision.HIGHEST)
                  + b_ref[...])


def _ag_stage(x, w, b2d):
    n, d = x.shape
    bm = 512
    return pl.pallas_call(
        _ag_kernel,
        grid=(pl.cdiv(n, bm),),
        in_specs=[
            pl.BlockSpec((bm, d), lambda i: (i, 0)),
            pl.BlockSpec((2 * d, d), lambda i: (0, 0)),
            pl.BlockSpec((1, d), lambda i: (0, 0)),
        ],
        out_specs=[
            pl.BlockSpec((bm, d), lambda i: (i, 0)),
            pl.BlockSpec((bm, d), lambda i: (i, 0)),
        ],
        out_shape=[
            jax.ShapeDtypeStruct((n, d), jnp.float32),
            jax.ShapeDtypeStruct((n, d), jnp.float32),
        ],
    )(x, w, b2d)


def _knn_tc_kernel(row0, n_hi, x_ref, xt_ref, idx_ref):
    i = pl.program_id(0)
    npad = xt_ref.shape[1]
    nchunks = npad // 128
    xb = x_ref[...]
    xt = xt_ref[...]

    sq_j = jnp.sum(xt * xt, axis=0, keepdims=True)
    dot = lax.dot_general(
        xb, xt, (((1,), (0,)), ((), ())),
        preferred_element_type=jnp.float32,
        precision=lax.Precision.DEFAULT)

    row_ids = row0 + i * BM + lax.broadcasted_iota(jnp.int32, (BM, 1), 0)

    lane = lax.broadcasted_iota(jnp.int32, (_SB, 128), 1)
    idx_parts, bad_parts = [], []
    for s in range(BM // _SB):
        r0 = s * _SB
        rids_s = row_ids[r0:r0 + _SB, :]
        vals = [jnp.full((_SB, 128), _BIG, jnp.float32) for _ in range(_LDEPTH)]
        chks = [jnp.zeros((_SB, 128), jnp.int32) for _ in range(_LDEPTH)]
        for c in range(nchunks):
            cs = c * 128
            v = sq_j[:, cs:cs + 128] - 2.0 * dot[r0:r0 + _SB, cs:cs + 128]
            lt = [v < vals[t] for t in range(_LDEPTH)]
            cc = jnp.full((_SB, 128), c, jnp.int32)
            new_vals, new_chks = [], []
            for t in range(_LDEPTH - 1, 0, -1):
                new_vals.append(jnp.where(lt[t],
                                          jnp.where(lt[t - 1], vals[t - 1], v),
                                          vals[t]))
                new_chks.append(jnp.where(lt[t],
                                          jnp.where(lt[t - 1], chks[t - 1], cc),
                                          chks[t]))
            new_vals.append(jnp.where(lt[0], v, vals[0]))
            new_chks.append(jnp.where(lt[0], cc, chks[0]))
            vals = new_vals[::-1]
            chks = new_chks[::-1]
        cv = jnp.concatenate(vals, axis=1)
        cid = jnp.concatenate([chks[t] * 128 + lane for t in range(_LDEPTH)],
                              axis=1)
        cols = []
        m = None
        for _ in range(K + 1):
            m = jnp.min(cv, axis=1, keepdims=True)
            sel = cv == m
            candid = jnp.where(sel, cid, _BIGI)
            j = jnp.min(candid, axis=1, keepdims=True)
            cols.append(j)
            cv = jnp.where(sel & (cid == j), _BIG, cv)
        found = cols[0] == rids_s
        outs = []
        for t in range(K):
            outs.append(jnp.where(found, cols[t + 1], cols[t]))
            if t + 1 < K:
                found = found | (cols[t + 1] == rids_s)
        idx_parts.append(jnp.concatenate(outs, axis=1))
        bad = jnp.max(jnp.where(vals[_LDEPTH - 1] <= m, 1, 0),
                      axis=1, keepdims=True)
        bad_parts.append(bad)
    idx_cand = jnp.concatenate(idx_parts, axis=0)
    badv = jnp.concatenate(bad_parts, axis=0)

    def _fallback():
        col_ids = lax.broadcasted_iota(jnp.int32, (BM, npad), 1)
        d2 = sq_j - 2.0 * dot
        d2 = jnp.where(col_ids == row_ids, _BIG, d2)
        return _extract16(d2, npad)

    ok = jnp.all((badv == 0) | (row_ids >= n_hi))
    idx_ref[...] = lax.cond(ok, lambda: idx_cand, _fallback)


def _knn_stage(x_part, xt_pad, row0):
    nrows, d = x_part.shape
    npad = xt_pad.shape[1]
    grid = (pl.cdiv(nrows, BM),)
    return pl.pallas_call(
        functools.partial(_knn_tc_kernel, row0, row0 + nrows),
        grid=grid,
        in_specs=[
            pl.BlockSpec((BM, d), lambda i: (i, 0)),
            pl.BlockSpec((d, npad), lambda i: (0, 0)),
        ],
        out_specs=pl.BlockSpec((BM, K), lambda i: (i, 0)),
        out_shape=jax.ShapeDtypeStruct((nrows, K), jnp.int32),
    )(x_part, xt_pad)


def _gather_max_stage(idx2d, a_pad, g, n_pad, d):
    npw = n_pad // _NW
    nchunks = npw // _CH
    cstride = ((nchunks + 7) // 8) * 8
    mesh = plsc.VectorSubcoreMesh(core_axis_name="c", subcore_axis_name="s")

    @functools.partial(
        pl.kernel,
        mesh=mesh,
        out_type=jax.ShapeDtypeStruct((n_pad, d), jnp.float32),
        scratch_types=[
            pltpu.VMEM((cstride, _CH * K), jnp.int32),
            pltpu.VMEM((_CH * K, d), jnp.float32),
            pltpu.VMEM((_CH * K, d), jnp.float32),
            pltpu.VMEM((npw, d), jnp.float32),
            pltpu.VMEM((npw, d), jnp.float32),
            pltpu.SemaphoreType.DMA,
            pltpu.SemaphoreType.DMA,
        ],
    )
    def sck(idx_hbm, a_hbm, g_hbm, out_hbm, idx_v, g_v0, g_v1, a_v, o_v,
            sem0, sem1):
        wid = lax.axis_index("s") * _NC + lax.axis_index("c")
        base = wid * npw

        pltpu.sync_copy(idx_hbm.at[pl.ds(wid * cstride, cstride)], idx_v)
        pltpu.sync_copy(a_hbm.at[pl.ds(base, npw)], a_v)

        gbufs = (g_v0, g_v1)
        sems = (sem0, sem1)

        def start(ci, b):
            pltpu.make_async_copy(g_hbm.at[idx_v.at[ci]], gbufs[b],
                                  sems[b]).start()

        def compute(ci, b):
            g_v = gbufs[b]
            pltpu.make_async_copy(g_hbm.at[idx_v.at[ci]], g_v, sems[b]).wait()

            @pl.loop(0, _CH)
            def _(nn):
                r0 = nn * K
                orow = ci * _CH + nn
                for v in range(d // _L):
                    sl = pl.ds(v * _L, _L)
                    acc = jnp.maximum(g_v[r0, sl], g_v[r0 + 1, sl])
                    for r in range(2, K):
                        acc = jnp.maximum(acc, g_v[r0 + r, sl])
                    o_v[orow, sl] = jnp.maximum(a_v[orow, sl] + acc, 0.0)

        start(0, 0)
        start(1, 1)

        @pl.loop(0, nchunks // 2)
        def _(gg):
            c0 = gg * 2
            for b in range(2):
                ci = c0 + b
                compute(ci, b)

                @pl.when(ci + 2 < nchunks)
                def _():
                    start(ci + 2, b)

        pltpu.sync_copy(o_v, out_hbm.at[pl.ds(base, npw)])

    return sck(idx2d, a_pad, g)


def kernel(x, edge_index, edge_attr, W, b):
    del edge_index, edge_attr
    n, d = x.shape
    npad = ((n + 127) // 128) * 128
    xt_pad = jnp.pad(x.T, ((0, 0), (0, npad - n)), constant_values=1000.0)
    b2d = b.reshape(1, d)

    a, g = _ag_stage(x, W, b2d)

    n_pad = ((n + _NW * _CH - 1) // (_NW * _CH)) * (_NW * _CH)
    a_pad = jnp.pad(a, ((0, n_pad - n), (0, 0)))

    idx = _knn_stage(x, xt_pad, 0)
    nchunks = n_pad // _NW // _CH
    cstride = ((nchunks + 7) // 8) * 8
    idx2d = jnp.pad(idx.reshape(-1), (0, (n_pad - n) * K)).reshape(
        _NW, nchunks, _CH * K)
    idx2d = jnp.pad(idx2d, ((0, 0), (0, cstride - nchunks),
                            (0, 0))).reshape(-1, _CH * K)
    out_pad = _gather_max_stage(idx2d, a_pad, g, n_pad, d)
    return out_pad[:n]
